# baseline (speedup 1.0000x reference)
import functools

import jax
import jax.numpy as jnp
from jax import lax
from jax.experimental import pallas as pl
from jax.experimental.pallas import tpu as pltpu
from jax.experimental.pallas import tpu_sc as plsc

N_IMG = 8
P = 256
ROWS_W = 28
COLS = 112
T_W = ROWS_W * COLS
TV = T_W // 16
IMG_STRIDE = 224 * 224
BIG = 3.0e38


def _shuf(v, idx):
    dnums = lax.GatherDimensionNumbers(
        offset_dims=(), collapsed_slice_dims=(0,), start_index_map=(0,))
    return lax.gather(v, idx.reshape(16, 1), dnums, slice_sizes=(1,),
                      mode=lax.GatherScatterMode.PROMISE_IN_BOUNDS)


def _lane_bcast(v, j):
    return _shuf(v, jnp.full((16,), j, jnp.int32))


def _allreduce(v, op):
    for sh in (1, 2, 4, 8):
        idx = lax.iota(jnp.int32, 16) ^ sh
        v = op(v, _shuf(v, idx))
    return v


def _chamfer_sc(bins_pad, depth_flat):
    mesh = plsc.VectorSubcoreMesh(core_axis_name="c", subcore_axis_name="s")

    @functools.partial(
        pl.kernel,
        out_type=jax.ShapeDtypeStruct((2, 16), jnp.float32),
        mesh=mesh,
        scratch_types=[
            pltpu.VMEM((T_W,), jnp.float32),
            pltpu.VMEM((264,), jnp.float32),
            pltpu.VMEM((P,), jnp.float32),
            pltpu.VMEM((T_W,), jnp.float32),
            pltpu.VMEM((P * 16,), jnp.float32),
            pltpu.VMEM((4 * P * 16,), jnp.float32),
            pltpu.VMEM((64,), jnp.float32),
            pltpu.VMEM((16,), jnp.float32),
            pltpu.VMEM_SHARED((16 * P * 16,), jnp.float32),
            pltpu.VMEM_SHARED((256,), jnp.float32),
            pltpu.VMEM_SHARED((256,), jnp.float32),
            pltpu.SemaphoreType.DMA,
        ],
    )
    def k(bins_hbm, depth_hbm, out_hbm, yv, binv, cent, ymin, xtab, cmb,
          ysum4, stage16, xtab_sh, ysum_sh, loss_sh, sem):
        c = lax.axis_index("c")
        s = lax.axis_index("s")
        n = c * 4 + s // 4
        q = s % 4

        base = n * IMG_STRIDE + 112 * 224 + 112 + q * (ROWS_W * 224)
        base = pl.multiple_of(base, 8)
        cps = []
        for r in range(ROWS_W):
            off = pl.multiple_of(base + r * 224, 8)
            cps.append(pltpu.async_copy(
                depth_hbm.at[pl.ds(off, COLS)],
                yv.at[pl.ds(r * COLS, COLS)], sem))
        boff = pl.multiple_of(n * 264, 8)
        cps.append(pltpu.async_copy(
            bins_hbm.at[pl.ds(boff, 264)], binv, sem))
        for cp in cps:
            cp.wait()

        for i in range(P // 16):
            e0 = binv[pl.ds(i * 16, 16)]
            e1 = binv[pl.ds(i * 16 + 1, 16)]
            cent[pl.ds(i * 16, 16)] = (e0 + e1) * jnp.float32(0.5)

        for chunk in range(P // 16):
            cv = cent[pl.ds(chunk * 16, 16)]
            cb = [_lane_bcast(cv, j) for j in range(16)]

            def body(t, xaccs, _chunk=chunk, _cb=cb):
                o = t * 16
                yvv = yv[pl.ds(o, 16)]
                if _chunk == 0:
                    ym = jnp.full((16,), BIG, jnp.float32)
                else:
                    ym = ymin[pl.ds(o, 16)]
                out = []
                for j in range(16):
                    d = yvv - _cb[j]
                    d = d * d
                    out.append(jnp.minimum(xaccs[j], d))
                    ym = jnp.minimum(ym, d)
                ymin[pl.ds(o, 16)] = ym
                return out

            xaccs = lax.fori_loop(
                0, TV, body, [jnp.full((16,), BIG, jnp.float32)] * 16,
                unroll=4)
            for j in range(16):
                xtab[pl.ds((chunk * 16 + j) * 16, 16)] = xaccs[j]

        syv = lax.fori_loop(
            0, TV, lambda t, a: a + ymin[pl.ds(t * 16, 16)],
            jnp.zeros((16,), jnp.float32), unroll=4)
        stage16[:] = syv
        soff = pl.multiple_of(s * 16, 8)
        pltpu.sync_copy(stage16, ysum_sh.at[pl.ds(soff, 16)])
        xoff = pl.multiple_of(s * (P * 16), 8)
        pltpu.sync_copy(xtab, xtab_sh.at[pl.ds(xoff, P * 16)])
        plsc.subcore_barrier()

        @pl.when(s % 4 == 0)
        def _():
            for kk in range(4):
                xo = pl.multiple_of((s + kk) * (P * 16), 8)
                pltpu.sync_copy(xtab_sh.at[pl.ds(xo, P * 16)],
                                cmb.at[pl.ds(kk * (P * 16), P * 16)])
                yo = pl.multiple_of((s + kk) * 16, 8)
                pltpu.sync_copy(ysum_sh.at[pl.ds(yo, 16)],
                                ysum4.at[pl.ds(kk * 16, 16)])

            def xbody(ci, acc):
                o = ci * 16
                r01 = jnp.minimum(cmb[pl.ds(o, 16)],
                                  cmb[pl.ds(o + P * 16, 16)])
                r23 = jnp.minimum(cmb[pl.ds(o + 2 * P * 16, 16)],
                                  cmb[pl.ds(o + 3 * P * 16, 16)])
                return acc + _allreduce(jnp.minimum(r01, r23), jnp.minimum)

            sx = lax.fori_loop(0, P, xbody, jnp.zeros((16,), jnp.float32))
            ysv = ((ysum4[pl.ds(0, 16)] + ysum4[pl.ds(16, 16)])
                   + (ysum4[pl.ds(32, 16)] + ysum4[pl.ds(48, 16)]))
            sy = _allreduce(ysv, jnp.add)
            contrib = (sx * jnp.float32(1.0 / P)
                       + sy * jnp.float32(1.0 / (4 * T_W))) \
                      * jnp.float32(1.0 / N_IMG)
            stage16[:] = contrib
            pltpu.sync_copy(stage16, loss_sh.at[pl.ds(soff, 16)])

        plsc.subcore_barrier()

        @pl.when(s == 0)
        def _():
            for kk in range(4):
                pltpu.sync_copy(loss_sh.at[pl.ds(kk * 64, 16)],
                                ysum4.at[pl.ds(kk * 16, 16)])
            tot = ((ysum4[pl.ds(0, 16)] + ysum4[pl.ds(16, 16)])
                   + (ysum4[pl.ds(32, 16)] + ysum4[pl.ds(48, 16)]))
            stage16[:] = tot
            pltpu.sync_copy(stage16, out_hbm.at[c])

    return k(bins_pad, depth_flat)


def kernel(bins, target_depth_maps):
    edges = bins.reshape(N_IMG, 257)
    bins_pad = jnp.pad(edges, ((0, 0), (0, 7)))
    bins_flat = bins_pad.reshape(-1)
    depth_flat = target_depth_maps.reshape(-1)
    out = _chamfer_sc(bins_flat, depth_flat)
    return out[0, 0] + out[1, 0]


# speedup vs baseline: 1.0845x; 1.0845x over previous
import functools

import jax
import jax.numpy as jnp
from jax import lax
from jax.experimental import pallas as pl
from jax.experimental.pallas import tpu as pltpu
from jax.experimental.pallas import tpu_sc as plsc

N_IMG = 8
P = 256
ROWS_W = 28
COLS = 112
T_W = ROWS_W * COLS
TV = T_W // 16
IMG_STRIDE = 224 * 224
BIG = 3.0e38


def _shuf(v, idx):
    dnums = lax.GatherDimensionNumbers(
        offset_dims=(), collapsed_slice_dims=(0,), start_index_map=(0,))
    return lax.gather(v, idx.reshape(16, 1), dnums, slice_sizes=(1,),
                      mode=lax.GatherScatterMode.PROMISE_IN_BOUNDS)


def _lane_bcast(v, j):
    return _shuf(v, jnp.full((16,), j, jnp.int32))


def _allreduce(v, op):
    for sh in (1, 2, 4, 8):
        idx = lax.iota(jnp.int32, 16) ^ sh
        v = op(v, _shuf(v, idx))
    return v


def _chamfer_sc(bins_pad, depth_flat):
    mesh = plsc.VectorSubcoreMesh(core_axis_name="c", subcore_axis_name="s")

    @functools.partial(
        pl.kernel,
        out_type=jax.ShapeDtypeStruct((2, 16), jnp.float32),
        mesh=mesh,
        scratch_types=[
            pltpu.VMEM((T_W,), jnp.float32),
            pltpu.VMEM((264,), jnp.float32),
            pltpu.VMEM((P,), jnp.float32),
            pltpu.VMEM((T_W,), jnp.float32),
            pltpu.VMEM((P * 16,), jnp.float32),
            pltpu.VMEM((4 * P * 16,), jnp.float32),
            pltpu.VMEM((64,), jnp.float32),
            pltpu.VMEM((16,), jnp.float32),
            pltpu.VMEM_SHARED((16 * P * 16,), jnp.float32),
            pltpu.VMEM_SHARED((256,), jnp.float32),
            pltpu.VMEM_SHARED((256,), jnp.float32),
            pltpu.SemaphoreType.DMA,
        ],
    )
    def k(bins_hbm, depth_hbm, out_hbm, yv, binv, cent, ymin, xtab, cmb,
          ysum4, stage16, xtab_sh, ysum_sh, loss_sh, sem):
        c = lax.axis_index("c")
        s = lax.axis_index("s")
        n = c * 4 + s // 4
        q = s % 4

        base = n * IMG_STRIDE + 112 * 224 + 112 + q * (ROWS_W * 224)
        base = pl.multiple_of(base, 8)
        cps = []
        for r in range(ROWS_W):
            off = pl.multiple_of(base + r * 224, 8)
            cps.append(pltpu.async_copy(
                depth_hbm.at[pl.ds(off, COLS)],
                yv.at[pl.ds(r * COLS, COLS)], sem))
        boff = pl.multiple_of(n * 264, 8)
        cps.append(pltpu.async_copy(
            bins_hbm.at[pl.ds(boff, 264)], binv, sem))
        for cp in cps:
            cp.wait()

        for i in range(P // 16):
            e0 = binv[pl.ds(i * 16, 16)]
            e1 = binv[pl.ds(i * 16 + 1, 16)]
            cent[pl.ds(i * 16, 16)] = (e0 + e1) * jnp.float32(0.5)

        for chunk in range(P // 16):
            cv = cent[pl.ds(chunk * 16, 16)]

            def body(t, xaccs, _chunk=chunk, _cv=cv):
                o = t * 16
                yvv = yv[pl.ds(o, 16)]
                if _chunk == 0:
                    ym = jnp.full((16,), BIG, jnp.float32)
                else:
                    ym = ymin[pl.ds(o, 16)]
                out = []
                ds = []
                for j in range(16):
                    d = yvv - _lane_bcast(_cv, j)
                    d = d * d
                    out.append(jnp.minimum(xaccs[j], d))
                    ds.append(d)
                while len(ds) > 1:
                    ds = [jnp.minimum(ds[i], ds[i + 1])
                          for i in range(0, len(ds), 2)]
                ymin[pl.ds(o, 16)] = jnp.minimum(ym, ds[0])
                return out

            xaccs = lax.fori_loop(
                0, TV, body, [jnp.full((16,), BIG, jnp.float32)] * 16,
                unroll=2)
            for j in range(16):
                xtab[pl.ds((chunk * 16 + j) * 16, 16)] = xaccs[j]

        syv = lax.fori_loop(
            0, TV, lambda t, a: a + ymin[pl.ds(t * 16, 16)],
            jnp.zeros((16,), jnp.float32), unroll=4)
        stage16[:] = syv
        soff = pl.multiple_of(s * 16, 8)
        pltpu.sync_copy(stage16, ysum_sh.at[pl.ds(soff, 16)])
        xoff = pl.multiple_of(s * (P * 16), 8)
        pltpu.sync_copy(xtab, xtab_sh.at[pl.ds(xoff, P * 16)])
        plsc.subcore_barrier()

        @pl.when(s % 4 == 0)
        def _():
            for kk in range(4):
                xo = pl.multiple_of((s + kk) * (P * 16), 8)
                pltpu.sync_copy(xtab_sh.at[pl.ds(xo, P * 16)],
                                cmb.at[pl.ds(kk * (P * 16), P * 16)])
                yo = pl.multiple_of((s + kk) * 16, 8)
                pltpu.sync_copy(ysum_sh.at[pl.ds(yo, 16)],
                                ysum4.at[pl.ds(kk * 16, 16)])

            def xbody(ci, acc):
                o = ci * 16
                r01 = jnp.minimum(cmb[pl.ds(o, 16)],
                                  cmb[pl.ds(o + P * 16, 16)])
                r23 = jnp.minimum(cmb[pl.ds(o + 2 * P * 16, 16)],
                                  cmb[pl.ds(o + 3 * P * 16, 16)])
                return acc + _allreduce(jnp.minimum(r01, r23), jnp.minimum)

            sx = lax.fori_loop(0, P, xbody, jnp.zeros((16,), jnp.float32))
            ysv = ((ysum4[pl.ds(0, 16)] + ysum4[pl.ds(16, 16)])
                   + (ysum4[pl.ds(32, 16)] + ysum4[pl.ds(48, 16)]))
            sy = _allreduce(ysv, jnp.add)
            contrib = (sx * jnp.float32(1.0 / P)
                       + sy * jnp.float32(1.0 / (4 * T_W))) \
                      * jnp.float32(1.0 / N_IMG)
            stage16[:] = contrib
            pltpu.sync_copy(stage16, loss_sh.at[pl.ds(soff, 16)])

        plsc.subcore_barrier()

        @pl.when(s == 0)
        def _():
            for kk in range(4):
                pltpu.sync_copy(loss_sh.at[pl.ds(kk * 64, 16)],
                                ysum4.at[pl.ds(kk * 16, 16)])
            tot = ((ysum4[pl.ds(0, 16)] + ysum4[pl.ds(16, 16)])
                   + (ysum4[pl.ds(32, 16)] + ysum4[pl.ds(48, 16)]))
            stage16[:] = tot
            pltpu.sync_copy(stage16, out_hbm.at[c])

    return k(bins_pad, depth_flat)


def kernel(bins, target_depth_maps):
    edges = bins.reshape(N_IMG, 257)
    bins_pad = jnp.pad(edges, ((0, 0), (0, 7)))
    bins_flat = bins_pad.reshape(-1)
    depth_flat = target_depth_maps.reshape(-1)
    out = _chamfer_sc(bins_flat, depth_flat)
    return out[0, 0] + out[1, 0]


# speedup vs baseline: 1.2180x; 1.1231x over previous
import functools

import jax
import jax.numpy as jnp
from jax import lax
from jax.experimental import pallas as pl
from jax.experimental.pallas import tpu as pltpu
from jax.experimental.pallas import tpu_sc as plsc

N_IMG = 8
P = 256
ROWS_W = 28
COLS = 112
T_W = ROWS_W * COLS
TV = T_W // 16
IMG_STRIDE = 224 * 224
BIG = 3.0e38


def _shuf(v, idx):
    dnums = lax.GatherDimensionNumbers(
        offset_dims=(), collapsed_slice_dims=(0,), start_index_map=(0,))
    return lax.gather(v, idx.reshape(16, 1), dnums, slice_sizes=(1,),
                      mode=lax.GatherScatterMode.PROMISE_IN_BOUNDS)


def _lane_bcast(v, j):
    return _shuf(v, jnp.full((16,), j, jnp.int32))


def _allreduce(v, op):
    for sh in (1, 2, 4, 8):
        idx = lax.iota(jnp.int32, 16) ^ sh
        v = op(v, _shuf(v, idx))
    return v


def _chamfer_sc(bins_pad, depth_flat):
    mesh = plsc.VectorSubcoreMesh(core_axis_name="c", subcore_axis_name="s")

    @functools.partial(
        pl.kernel,
        out_type=jax.ShapeDtypeStruct((2, 16), jnp.float32),
        mesh=mesh,
        scratch_types=[
            pltpu.VMEM((T_W,), jnp.float32),
            pltpu.VMEM((264,), jnp.float32),
            pltpu.VMEM((P,), jnp.float32),
            pltpu.VMEM((T_W,), jnp.float32),
            pltpu.VMEM((P * 16,), jnp.float32),
            pltpu.VMEM((4 * P * 16,), jnp.float32),
            pltpu.VMEM((64,), jnp.float32),
            pltpu.VMEM((16,), jnp.float32),
            pltpu.VMEM_SHARED((16 * P * 16,), jnp.float32),
            pltpu.VMEM_SHARED((256,), jnp.float32),
            pltpu.VMEM_SHARED((256,), jnp.float32),
            pltpu.SemaphoreType.DMA,
        ],
    )
    def k(bins_hbm, depth_hbm, out_hbm, yv, binv, cent, ymin, xtab, cmb,
          ysum4, stage16, xtab_sh, ysum_sh, loss_sh, sem):
        c = lax.axis_index("c")
        s = lax.axis_index("s")
        n = c * 4 + s // 4
        q = s % 4

        base = n * IMG_STRIDE + 112 * 224 + 112 + q * (ROWS_W * 224)
        base = pl.multiple_of(base, 8)
        cps = []
        for r in range(ROWS_W):
            off = pl.multiple_of(base + r * 224, 8)
            cps.append(pltpu.async_copy(
                depth_hbm.at[pl.ds(off, COLS)],
                yv.at[pl.ds(r * COLS, COLS)], sem))
        boff = pl.multiple_of(n * 264, 8)
        cps.append(pltpu.async_copy(
            bins_hbm.at[pl.ds(boff, 264)], binv, sem))
        for cp in cps:
            cp.wait()

        for i in range(P // 16):
            e0 = binv[pl.ds(i * 16, 16)]
            e1 = binv[pl.ds(i * 16 + 1, 16)]
            cent[pl.ds(i * 16, 16)] = (e0 + e1) * jnp.float32(0.5)

        for chunk in range(P // 16):
            cv = cent[pl.ds(chunk * 16, 16)]

            def body(t, xaccs, _chunk=chunk, _cv=cv):
                o = t * 16
                yvv = yv[pl.ds(o, 16)]
                if _chunk == 0:
                    ym = jnp.full((16,), BIG, jnp.float32)
                else:
                    ym = ymin[pl.ds(o, 16)]
                out = []
                ds = []
                for j in range(16):
                    d = yvv - _lane_bcast(_cv, j)
                    d = d * d
                    out.append(jnp.minimum(xaccs[j], d))
                    ds.append(d)
                while len(ds) > 1:
                    ds = [jnp.minimum(ds[i], ds[i + 1])
                          for i in range(0, len(ds), 2)]
                ymin[pl.ds(o, 16)] = jnp.minimum(ym, ds[0])
                return out

            xaccs = lax.fori_loop(
                0, TV, body, [jnp.full((16,), BIG, jnp.float32)] * 16)
            for j in range(16):
                xtab[pl.ds((chunk * 16 + j) * 16, 16)] = xaccs[j]

        syv = lax.fori_loop(
            0, TV, lambda t, a: a + ymin[pl.ds(t * 16, 16)],
            jnp.zeros((16,), jnp.float32), unroll=4)
        stage16[:] = syv
        soff = pl.multiple_of(s * 16, 8)
        pltpu.sync_copy(stage16, ysum_sh.at[pl.ds(soff, 16)])
        xoff = pl.multiple_of(s * (P * 16), 8)
        pltpu.sync_copy(xtab, xtab_sh.at[pl.ds(xoff, P * 16)])
        plsc.subcore_barrier()

        @pl.when(s % 4 == 0)
        def _():
            for kk in range(4):
                xo = pl.multiple_of((s + kk) * (P * 16), 8)
                pltpu.sync_copy(xtab_sh.at[pl.ds(xo, P * 16)],
                                cmb.at[pl.ds(kk * (P * 16), P * 16)])
                yo = pl.multiple_of((s + kk) * 16, 8)
                pltpu.sync_copy(ysum_sh.at[pl.ds(yo, 16)],
                                ysum4.at[pl.ds(kk * 16, 16)])

            def xbody(ci, acc):
                o = ci * 16
                r01 = jnp.minimum(cmb[pl.ds(o, 16)],
                                  cmb[pl.ds(o + P * 16, 16)])
                r23 = jnp.minimum(cmb[pl.ds(o + 2 * P * 16, 16)],
                                  cmb[pl.ds(o + 3 * P * 16, 16)])
                return acc + _allreduce(jnp.minimum(r01, r23), jnp.minimum)

            sx = lax.fori_loop(0, P, xbody, jnp.zeros((16,), jnp.float32))
            ysv = ((ysum4[pl.ds(0, 16)] + ysum4[pl.ds(16, 16)])
                   + (ysum4[pl.ds(32, 16)] + ysum4[pl.ds(48, 16)]))
            sy = _allreduce(ysv, jnp.add)
            contrib = (sx * jnp.float32(1.0 / P)
                       + sy * jnp.float32(1.0 / (4 * T_W))) \
                      * jnp.float32(1.0 / N_IMG)
            stage16[:] = contrib
            pltpu.sync_copy(stage16, loss_sh.at[pl.ds(soff, 16)])

        plsc.subcore_barrier()

        @pl.when(s == 0)
        def _():
            for kk in range(4):
                pltpu.sync_copy(loss_sh.at[pl.ds(kk * 64, 16)],
                                ysum4.at[pl.ds(kk * 16, 16)])
            tot = ((ysum4[pl.ds(0, 16)] + ysum4[pl.ds(16, 16)])
                   + (ysum4[pl.ds(32, 16)] + ysum4[pl.ds(48, 16)]))
            stage16[:] = tot
            pltpu.sync_copy(stage16, out_hbm.at[c])

    return k(bins_pad, depth_flat)


def kernel(bins, target_depth_maps):
    edges = bins.reshape(N_IMG, 257)
    bins_pad = jnp.pad(edges, ((0, 0), (0, 7)))
    bins_flat = bins_pad.reshape(-1)
    depth_flat = target_depth_maps.reshape(-1)
    out = _chamfer_sc(bins_flat, depth_flat)
    return out[0, 0] + out[1, 0]
